# baseline (speedup 1.0000x reference)
import functools
import math

import jax
import jax.numpy as jnp
from jax import lax
from jax.experimental import pallas as pl
from jax.experimental.pallas import tpu as pltpu
from jax.experimental.pallas import tpu_sc as plsc

D_MODEL = 64
SCALE = math.sqrt(D_MODEL)
LANES = 16
NUM_CORES = 2
NUM_SUBCORES = 16
NUM_WORKERS = NUM_CORES * NUM_SUBCORES


@functools.lru_cache(maxsize=None)
def _build(n_total: int, vocab8: int, d: int, chunk: int):
    per_w = n_total // NUM_WORKERS
    n_chunks = per_w // chunk
    assert n_chunks % 2 == 0 and n_chunks >= 4
    n_groups = n_chunks // 2
    slices_per_row = d // LANES

    mesh = plsc.VectorSubcoreMesh(core_axis_name="c", subcore_axis_name="s")

    @functools.partial(
        pl.kernel,
        out_type=jax.ShapeDtypeStruct((n_total, d), jnp.float32),
        mesh=mesh,
        scratch_types=[
            pltpu.VMEM((per_w + LANES,), jnp.int32),
            pltpu.VMEM((2, chunk, d), jnp.float32),
            pltpu.VMEM((2, chunk, d), jnp.float32),
            pltpu.SemaphoreType.DMA,
            pltpu.SemaphoreType.DMA,
            pltpu.SemaphoreType.DMA,
            pltpu.SemaphoreType.DMA,
        ],
        compiler_params=pltpu.CompilerParams(use_tc_tiling_on_sc=True),
    )
    def emb_kernel(x_hbm, table_hbm, out_hbm, idx_v, gbuf, sbuf,
                   gsem0, gsem1, osem0, osem1):
        gsems = (gsem0, gsem1)
        osems = (osem0, osem1)
        wid = lax.axis_index("s") * NUM_CORES + lax.axis_index("c")
        base = wid * per_w
        pltpu.sync_copy(x_hbm.at[pl.ds(base, per_w)],
                        idx_v.at[pl.ds(0, per_w)])

        def fire_gather(c, b):
            cb = c * chunk

            @pl.loop(0, chunk // LANES)
            def issue(g):
                vec = idx_v[pl.ds(cb + g * LANES, LANES)]
                for l in range(LANES):
                    iv = vec[l]
                    pltpu.async_copy(
                        table_hbm.at[iv >> 3, iv & 7, :],
                        gbuf.at[b, g * LANES + l], gsems[b])

        fire_gather(0, 0)
        fire_gather(1, 1)

        @pl.loop(0, n_groups)
        def group(g):
            for b in range(2):
                c = g * 2 + b
                cb = c * chunk
                pltpu.make_async_copy(
                    out_hbm.at[pl.ds(0, chunk)], gbuf.at[b],
                    gsems[b]).wait()
                @pl.when(g > 0)
                def _():
                    pltpu.make_async_copy(
                        sbuf.at[b], out_hbm.at[pl.ds(0, chunk)],
                        osems[b]).wait()

                @pl.when(g < n_groups - 1)
                def _():
                    nb = (c + 2) * chunk

                    @plsc.parallel_loop(0, chunk, unroll=2)
                    def scale_and_fire(r):
                        iv = idx_v[pl.ds(nb + r, LANES)][0]
                        for j in range(slices_per_row):
                            sl = pl.ds(j * LANES, LANES)
                            sbuf[b, r, sl] = gbuf[b, r, sl] * SCALE
                        pltpu.async_copy(
                            table_hbm.at[iv >> 3, iv & 7, :],
                            gbuf.at[b, r], gsems[b])

                @pl.when(g == n_groups - 1)
                def _():
                    @plsc.parallel_loop(0, chunk, unroll=4)
                    def scale(r):
                        for j in range(slices_per_row):
                            sl = pl.ds(j * LANES, LANES)
                            sbuf[b, r, sl] = gbuf[b, r, sl] * SCALE

                pltpu.async_copy(
                    sbuf.at[b], out_hbm.at[pl.ds(base + cb, chunk)],
                    osems[b])

        for b in range(2):
            pltpu.make_async_copy(
                sbuf.at[b], out_hbm.at[pl.ds(0, chunk)], osems[b]).wait()

    return emb_kernel


def _pick_chunk(per_w: int, target: int) -> int:
    best = None
    for c in range(LANES, per_w + 1, LANES):
        if per_w % c or (per_w // c) % 2 or per_w // c < 4:
            continue
        if best is None or abs(c - target) < abs(best - target):
            best = c
    return best if best is not None else per_w


def kernel(x, table):
    b_dim, s_dim = x.shape
    xf = x.T.reshape(-1).astype(jnp.int32)
    n_total = xf.shape[0]
    vocab, d = table.shape
    assert vocab % 8 == 0 and n_total % NUM_WORKERS == 0
    t3 = table.reshape(vocab // 8, 8, d)
    per_w = n_total // NUM_WORKERS
    chunk = _pick_chunk(per_w, 160)
    out = _build(n_total, vocab // 8, d, chunk)(xf, t3)
    return out.reshape(s_dim, b_dim, d).transpose(1, 0, 2)
